# baseline (speedup 1.0000x reference)
import functools
import jax
import jax.numpy as jnp
from jax import lax
from jax.experimental import pallas as pl
from jax.experimental.pallas import tpu as pltpu
from jax.experimental.pallas import tpu_sc as plsc

VOCAB = 100004
EMBED_DIM = 128
BATCH = 4096
HIST_LEN = 200

_TOTAL = BATCH * HIST_LEN
_CHUNK = 256
_NW = 32
_PER_W = _TOTAL // _NW
_NCHUNK = _PER_W // _CHUNK


def _gather_body(idx_hbm, table_hbm, out_hbm, idx_v, rows0, rows1, sem0, sem1):
    wid = lax.axis_index("s") * 2 + lax.axis_index("c")
    base = wid * _PER_W

    pltpu.sync_copy(idx_hbm.at[pl.ds(base, _PER_W)], idx_v)

    rows = (rows0, rows1)
    sems = (sem0, sem1)

    def gather_start(g, b):
        pltpu.async_copy(
            table_hbm.at[idx_v.at[pl.ds(g * _CHUNK, _CHUNK)]], rows[b], sems[b]
        )

    def store_sync(g, b):
        pltpu.sync_copy(rows[b], out_hbm.at[pl.ds(base + g * _CHUNK, _CHUNK)])

    def wait_gather(b):
        pltpu.make_async_copy(
            table_hbm.at[idx_v.at[pl.ds(0, _CHUNK)]], rows[b], sems[b]
        ).wait()

    gather_start(0, 0)

    @pl.loop(0, _NCHUNK - 2, step=2)
    def _(g0):
        for b in range(2):
            g = g0 + b
            gather_start(g + 1, 1 - b)
            wait_gather(b)
            store_sync(g, b)

    g_tail = _NCHUNK - 2
    gather_start(g_tail + 1, 1)
    wait_gather(0)
    store_sync(g_tail, 0)
    wait_gather(1)
    store_sync(g_tail + 1, 1)


def kernel(input_indices, embedding_matrix):
    idx_flat = input_indices.reshape(_TOTAL)

    mesh = plsc.VectorSubcoreMesh(core_axis_name="c", subcore_axis_name="s")
    out_flat = pl.kernel(
        _gather_body,
        mesh=mesh,
        out_type=jax.ShapeDtypeStruct((_TOTAL, EMBED_DIM), jnp.float32),
        scratch_types=[
            pltpu.VMEM((_PER_W,), jnp.int32),
            pltpu.VMEM((_CHUNK, EMBED_DIM), jnp.float32),
            pltpu.VMEM((_CHUNK, EMBED_DIM), jnp.float32),
            pltpu.SemaphoreType.DMA,
            pltpu.SemaphoreType.DMA,
        ],
    )(idx_flat, embedding_matrix)

    return out_flat.reshape(BATCH, HIST_LEN, EMBED_DIM)


# speedup vs baseline: 1.0309x; 1.0309x over previous
import functools
import jax
import jax.numpy as jnp
from jax import lax
from jax.experimental import pallas as pl
from jax.experimental.pallas import tpu as pltpu
from jax.experimental.pallas import tpu_sc as plsc

VOCAB = 100004
EMBED_DIM = 128
BATCH = 4096
HIST_LEN = 200

_TOTAL = BATCH * HIST_LEN
_CHUNK = 256
_NW = 32
_PER_W = _TOTAL // _NW
_NCHUNK = _PER_W // _CHUNK


def _gather_body(idx_hbm, table_hbm, out_hbm, idx_v, rows0, rows1, shared,
                 sem0, sem1, semsh):
    wid = lax.axis_index("s") * 2 + lax.axis_index("c")
    sid = lax.axis_index("s")
    base = wid * _PER_W

    pltpu.sync_copy(idx_hbm.at[pl.ds(base, _PER_W)], idx_v)

    rows = (rows0, rows1)
    sems = (sem0, sem1)

    def gather_start(g, b):
        pltpu.async_copy(
            table_hbm.at[idx_v.at[pl.ds(g * _CHUNK, _CHUNK)]], rows[b], sems[b]
        )

    def store_sync(g, b):
        pltpu.sync_copy(rows[b], shared.at[sid])
        pltpu.async_copy(
            shared.at[sid], out_hbm.at[pl.ds(base + g * _CHUNK, _CHUNK)], semsh
        ).wait()

    def wait_gather(b):
        pltpu.make_async_copy(
            table_hbm.at[idx_v.at[pl.ds(0, _CHUNK)]], rows[b], sems[b]
        ).wait()

    gather_start(0, 0)

    @pl.loop(0, _NCHUNK - 2, step=2)
    def _(g0):
        for b in range(2):
            g = g0 + b
            gather_start(g + 1, 1 - b)
            wait_gather(b)
            store_sync(g, b)

    g_tail = _NCHUNK - 2
    gather_start(g_tail + 1, 1)
    wait_gather(0)
    store_sync(g_tail, 0)
    wait_gather(1)
    store_sync(g_tail + 1, 1)


def kernel(input_indices, embedding_matrix):
    idx_flat = input_indices.reshape(_TOTAL)

    mesh = plsc.VectorSubcoreMesh(core_axis_name="c", subcore_axis_name="s")
    out_flat = pl.kernel(
        _gather_body,
        mesh=mesh,
        out_type=jax.ShapeDtypeStruct((_TOTAL, EMBED_DIM), jnp.float32),
        scratch_types=[
            pltpu.VMEM((_PER_W,), jnp.int32),
            pltpu.VMEM((_CHUNK, EMBED_DIM), jnp.float32),
            pltpu.VMEM((_CHUNK, EMBED_DIM), jnp.float32),
            pltpu.VMEM_SHARED((16, _CHUNK, EMBED_DIM), jnp.float32),
            pltpu.SemaphoreType.DMA,
            pltpu.SemaphoreType.DMA,
            pltpu.SemaphoreType.DMA,
        ],
    )(idx_flat, embedding_matrix)

    return out_flat.reshape(BATCH, HIST_LEN, EMBED_DIM)
